# Initial kernel scaffold; baseline (speedup 1.0000x reference)
import jax
import jax.numpy as jnp
from jax.experimental import pallas as pl


def kernel(x, w, indices):
    raise NotImplementedError("write your pallas kernel here")


# speedup vs baseline: 5.5697x; 5.5697x over previous
import functools

import jax
import jax.numpy as jnp
from jax import lax
from jax.experimental import pallas as pl
from jax.experimental.pallas import tpu as pltpu
from jax.experimental.pallas import tpu_sc as plsc

M_OUT = 16384
NCORES = 2
NSUB = 16
GRP = 128
BC = 32


def _sc_spmm(xb, rows, cols, wv):
    nrounds, n_in, _ = xb.shape
    ngroups = rows.shape[1]
    rpc = nrounds // NCORES
    rows_per_tile = M_OUT // NSUB
    stage_per_tile = n_in // NSUB

    mesh = plsc.VectorSubcoreMesh(core_axis_name="c", subcore_axis_name="s")

    @functools.partial(
        pl.kernel,
        out_type=jax.ShapeDtypeStruct((nrounds, M_OUT, BC), jnp.float32),
        mesh=mesh,
        scratch_types=[
            pltpu.VMEM_SHARED((n_in, BC), jnp.float32),
            pltpu.VMEM_SHARED((M_OUT, BC), jnp.float32),
            pltpu.VMEM((ngroups, GRP), jnp.int32),
            pltpu.VMEM((ngroups, GRP), jnp.int32),
            pltpu.VMEM((ngroups, GRP), jnp.float32),
            pltpu.VMEM((GRP, BC), jnp.float32),
            pltpu.VMEM((GRP, BC), jnp.float32),
        ],
        compiler_params=pltpu.CompilerParams(use_tc_tiling_on_sc=False),
    )
    def k(xb_hbm, rows_hbm, cols_hbm, w_hbm, out_hbm,
          xs_sh, acc_sh, rows_v, cols_v, w_v, gbuf, zbuf):
        c = lax.axis_index("c")
        s = lax.axis_index("s")

        pltpu.sync_copy(rows_hbm.at[s], rows_v)
        pltpu.sync_copy(cols_hbm.at[s], cols_v)
        pltpu.sync_copy(w_hbm.at[s], w_v)

        zeros16 = jnp.zeros((16,), jnp.float32)

        def zb_body(i, carry):
            for cc in range(BC // 16):
                zbuf[i, pl.ds(cc * 16, 16)] = zeros16
            return carry

        lax.fori_loop(0, GRP, zb_body, 0)

        acc_lo = s * rows_per_tile
        stage_lo = s * stage_per_tile

        for r_local in range(rpc):
            r = c * rpc + r_local
            pltpu.sync_copy(
                xb_hbm.at[r, pl.ds(stage_lo, stage_per_tile)],
                xs_sh.at[pl.ds(stage_lo, stage_per_tile)],
            )
            def zero_body(z, carry):
                pltpu.sync_copy(
                    zbuf, acc_sh.at[pl.ds(acc_lo + z * GRP, GRP)]
                )
                return carry

            lax.fori_loop(0, rows_per_tile // GRP, zero_body, 0)
            plsc.subcore_barrier()

            def grp_body(g, carry):
                pltpu.sync_copy(xs_sh.at[rows_v.at[g]], gbuf)

                def scale_sub(t, inner):
                    w16 = w_v[g, pl.ds(t * 16, 16)]
                    for l in range(16):
                        i = t * 16 + l
                        wb = lax.broadcast(w16[l], (16,))
                        for cc in range(BC // 16):
                            sl = pl.ds(cc * 16, 16)
                            gbuf[i, sl] = gbuf[i, sl] * wb
                    return inner

                lax.fori_loop(0, GRP // 16, scale_sub, 0)
                pltpu.sync_copy(gbuf, acc_sh.at[cols_v.at[g]], add=True)
                return carry

            lax.fori_loop(0, ngroups, grp_body, 0)

            plsc.subcore_barrier()
            pltpu.sync_copy(
                acc_sh.at[pl.ds(acc_lo, rows_per_tile)],
                out_hbm.at[r, pl.ds(acc_lo, rows_per_tile)],
            )
            plsc.subcore_barrier()

    return k(xb, rows, cols, wv)


def kernel(x, w, indices):
    b, n_in = x.shape
    nnz = w.shape[0]
    nrounds = b // BC

    per_tile = GRP * NSUB
    ngroups = -(-nnz // per_tile)
    padded = ngroups * per_tile
    pad = padded - nnz

    rows = jnp.pad(indices[:, 0], (0, pad)).reshape(NSUB, ngroups, GRP)
    cols = jnp.pad(indices[:, 1], (0, pad)).reshape(NSUB, ngroups, GRP)
    wp = jnp.pad(w, (0, pad)).reshape(NSUB, ngroups, GRP)

    xb = x.reshape(nrounds, BC, n_in).transpose(0, 2, 1)
    outb = _sc_spmm(xb, rows, cols, wp)
    return outb.transpose(0, 2, 1).reshape(b, M_OUT)


# speedup vs baseline: 7.0081x; 1.2583x over previous
import functools

import jax
import jax.numpy as jnp
from jax import lax
from jax.experimental import pallas as pl
from jax.experimental.pallas import tpu as pltpu
from jax.experimental.pallas import tpu_sc as plsc

M_OUT = 16384
NCORES = 2
NSUB = 16
GRP = 128
BC = 32


def _sc_spmm(xb, rows, cols, wv):
    nrounds, n_in, _ = xb.shape
    ngroups = rows.shape[1]
    rpc = nrounds // NCORES
    rows_per_tile = M_OUT // NSUB
    stage_per_tile = n_in // NSUB

    mesh = plsc.VectorSubcoreMesh(core_axis_name="c", subcore_axis_name="s")

    @functools.partial(
        pl.kernel,
        out_type=jax.ShapeDtypeStruct((nrounds, M_OUT, BC), jnp.float32),
        mesh=mesh,
        scratch_types=[
            pltpu.VMEM_SHARED((n_in, BC), jnp.float32),
            pltpu.VMEM_SHARED((M_OUT, BC), jnp.float32),
            pltpu.VMEM((ngroups, GRP), jnp.int32),
            pltpu.VMEM((ngroups, GRP), jnp.int32),
            pltpu.VMEM((ngroups, GRP), jnp.float32),
            pltpu.VMEM((GRP, BC), jnp.float32),
            pltpu.VMEM((GRP, BC), jnp.float32),
            pltpu.VMEM((GRP, BC), jnp.float32),
            pltpu.SemaphoreType.DMA,
            pltpu.SemaphoreType.DMA,
            pltpu.SemaphoreType.DMA,
            pltpu.SemaphoreType.DMA,
        ],
        compiler_params=pltpu.CompilerParams(use_tc_tiling_on_sc=False),
    )
    def k(xb_hbm, rows_hbm, cols_hbm, w_hbm, out_hbm,
          xs_sh, acc_sh, rows_v, cols_v, w_v, gbuf0, gbuf1, zbuf,
          sg0, sg1, ss0, ss1):
        c = lax.axis_index("c")
        s = lax.axis_index("s")

        pltpu.sync_copy(rows_hbm.at[s], rows_v)
        pltpu.sync_copy(cols_hbm.at[s], cols_v)
        pltpu.sync_copy(w_hbm.at[s], w_v)

        zeros16 = jnp.zeros((16,), jnp.float32)

        def zb_body(i, carry):
            for cc in range(BC // 16):
                zbuf[i, pl.ds(cc * 16, 16)] = zeros16
            return carry

        lax.fori_loop(0, GRP, zb_body, 0)

        acc_lo = s * rows_per_tile
        stage_lo = s * stage_per_tile

        for r_local in range(rpc):
            r = c * rpc + r_local
            pltpu.sync_copy(
                xb_hbm.at[r, pl.ds(stage_lo, stage_per_tile)],
                xs_sh.at[pl.ds(stage_lo, stage_per_tile)],
            )
            def zero_body(z, carry):
                pltpu.sync_copy(
                    zbuf, acc_sh.at[pl.ds(acc_lo + z * GRP, GRP)]
                )
                return carry

            lax.fori_loop(0, rows_per_tile // GRP, zero_body, 0)
            plsc.subcore_barrier()

            def scale(buf, g):
                def scale_sub(t, inner):
                    w16 = w_v[g, pl.ds(t * 16, 16)]
                    for l in range(16):
                        i = t * 16 + l
                        wb = lax.broadcast(w16[l], (16,))
                        for cc in range(BC // 16):
                            sl = pl.ds(cc * 16, 16)
                            buf[i, sl] = buf[i, sl] * wb
                    return inner

                lax.fori_loop(0, GRP // 16, scale_sub, 0)

            pltpu.async_copy(xs_sh.at[rows_v.at[0]], gbuf0, sg0)
            pltpu.async_copy(xs_sh.at[rows_v.at[1]], gbuf1, sg1)

            def pair_body(p, carry):
                g0 = p * 2
                g1 = g0 + 1
                pltpu.make_async_copy(xs_sh.at[rows_v.at[g0]], gbuf0, sg0).wait()
                scale(gbuf0, g0)
                pltpu.async_copy(gbuf0, acc_sh.at[cols_v.at[g0]], ss0, add=True)

                pltpu.make_async_copy(xs_sh.at[rows_v.at[g1]], gbuf1, sg1).wait()
                scale(gbuf1, g1)
                pltpu.async_copy(gbuf1, acc_sh.at[cols_v.at[g1]], ss1, add=True)

                pltpu.make_async_copy(gbuf0, acc_sh.at[cols_v.at[g0]], ss0).wait()
                pltpu.async_copy(xs_sh.at[rows_v.at[g0 + 2]], gbuf0, sg0)
                pltpu.make_async_copy(gbuf1, acc_sh.at[cols_v.at[g1]], ss1).wait()
                pltpu.async_copy(xs_sh.at[rows_v.at[g1 + 2]], gbuf1, sg1)
                return carry

            lax.fori_loop(0, ngroups // 2 - 1, pair_body, 0)

            gl0 = ngroups - 2
            gl1 = ngroups - 1
            pltpu.make_async_copy(xs_sh.at[rows_v.at[gl0]], gbuf0, sg0).wait()
            scale(gbuf0, gl0)
            pltpu.async_copy(gbuf0, acc_sh.at[cols_v.at[gl0]], ss0, add=True)
            pltpu.make_async_copy(xs_sh.at[rows_v.at[gl1]], gbuf1, sg1).wait()
            scale(gbuf1, gl1)
            pltpu.async_copy(gbuf1, acc_sh.at[cols_v.at[gl1]], ss1, add=True)
            pltpu.make_async_copy(gbuf0, acc_sh.at[cols_v.at[gl0]], ss0).wait()
            pltpu.make_async_copy(gbuf1, acc_sh.at[cols_v.at[gl1]], ss1).wait()

            plsc.subcore_barrier()
            pltpu.sync_copy(
                acc_sh.at[pl.ds(acc_lo, rows_per_tile)],
                out_hbm.at[r, pl.ds(acc_lo, rows_per_tile)],
            )
            plsc.subcore_barrier()

    return k(xb, rows, cols, wv)


def kernel(x, w, indices):
    b, n_in = x.shape
    nnz = w.shape[0]
    nrounds = b // BC

    per_tile = GRP * NSUB
    ngroups = -(-nnz // per_tile)
    ngroups += ngroups % 2
    padded = ngroups * per_tile
    pad = padded - nnz

    rows = jnp.pad(indices[:, 0], (0, pad)).reshape(NSUB, ngroups, GRP)
    cols = jnp.pad(indices[:, 1], (0, pad)).reshape(NSUB, ngroups, GRP)
    wp = jnp.pad(w, (0, pad)).reshape(NSUB, ngroups, GRP)

    xb = x.reshape(nrounds, BC, n_in).transpose(0, 2, 1)
    outb = _sc_spmm(xb, rows, cols, wp)
    return outb.transpose(0, 2, 1).reshape(b, M_OUT)
